# Initial kernel scaffold; baseline (speedup 1.0000x reference)
import jax
import jax.numpy as jnp
from jax.experimental import pallas as pl


def kernel(byte_seq, target_byte_seq, emb_byte, enc_Wq, enc_Wk, enc_Wv, enc_Wo, enc_ff1, enc_ff2, proj_Wr, proj_Wi, cq_r, cq_i, ck_r, ck_i, cv_r, cv_i, co_r, co_i, c2r_W, dec_emb, dec_Wq, dec_Wk, dec_Wv, dec_Wo, dec_ff1, dec_ff2, out_W):
    raise NotImplementedError("write your pallas kernel here")


# speedup vs baseline: 1.4910x; 1.4910x over previous
import functools
import math

import jax
import jax.numpy as jnp
from jax.experimental import pallas as pl
from jax.experimental.pallas import tpu as pltpu

B = 8; S = 2048; T = 2048; H = 512; C = 512; NH = 8; DM = 768; L = 6; P = 128; V = 256; FF = 2048

CDT = jnp.bfloat16
ACC = jnp.float32

ENC_TB = 512
ENC_NPB = ENC_TB // P
DEC_TB = 1024
DH_E = H // NH
DH_D = DM // NH


def _mm(a, b):
    return jax.lax.dot_general(a.astype(CDT), b.astype(CDT),
                               (((1,), (0,)), ((), ())),
                               preferred_element_type=ACC)


def _mm_t(a, b):
    return jax.lax.dot_general(a.astype(CDT), b.astype(CDT),
                               (((1,), (1,)), ((), ())),
                               preferred_element_type=ACC)


def _mm_tt(a, b):
    return jax.lax.dot_general(a.astype(CDT), b.astype(CDT),
                               (((0,), (0,)), ((), ())),
                               preferred_element_type=ACC)


def _ln_f32(x):
    m = jnp.mean(x, axis=-1, keepdims=True)
    v = jnp.mean((x - m) * (x - m), axis=-1, keepdims=True)
    return (x - m) / jnp.sqrt(v + 1e-5)


def _gelu(x):
    c = math.sqrt(2.0 / math.pi)
    return 0.5 * x * (1.0 + jnp.tanh(c * (x + 0.044715 * (x * x * x))))


def _head_mask(width, dh, h):
    lane = jax.lax.broadcasted_iota(jnp.int32, (1, width), 1)
    return (lane // dh == h).astype(ACC)


def _softmax_rows(s):
    m = jnp.max(s, axis=-1, keepdims=True)
    e = jnp.exp(s - m)
    return e / jnp.sum(e, axis=-1, keepdims=True)


def _enc_kernel(ids_ref, emb_ref, wq_ref, wk_ref, wv_ref, wo_ref,
                ff1_ref, ff2_ref, out_ref):
    ids = ids_ref[0]
    oh = (jax.lax.broadcasted_iota(jnp.int32, (V, ENC_TB), 0) == ids)
    x = _mm_tt(oh.astype(CDT), emb_ref[...])
    h1 = _ln_f32(x)
    q = _mm(h1, wq_ref[...])
    k = _mm(h1, wk_ref[...])
    v = _mm(h1, wv_ref[...])
    qm = [q * _head_mask(H, DH_E, h) for h in range(NH)]
    vm = [v * _head_mask(H, DH_E, h) for h in range(NH)]
    scale = 1.0 / math.sqrt(DH_E)
    rows = []
    for p in range(ENC_NPB):
        sl = slice(p * P, (p + 1) * P)
        kp = k[sl]
        a_list = [_softmax_rows(_mm_t(qm[h][sl], kp) * scale) for h in range(NH)]
        A = jnp.concatenate(a_list, axis=1)
        Vhat = jnp.concatenate([vm[h][sl] for h in range(NH)], axis=0)
        rows.append(_mm(A, Vhat))
    attn = jnp.concatenate(rows, axis=0)
    h = x + _mm(attn, wo_ref[...])
    h2 = _ln_f32(h)
    h = h + _mm(_gelu(_mm(h2, ff1_ref[...])), ff2_ref[...])
    pooled = jnp.concatenate(
        [jnp.mean(h[p * P:(p + 1) * P], axis=0, keepdims=True)
         for p in range(ENC_NPB)], axis=0)
    out_ref[0] = pooled


def _cplx_kernel(pr_ref, pwr_ref, pwi_ref, cqr_ref, cqi_ref, ckr_ref, cki_ref,
                 cvr_ref, cvi_ref, cor_ref, coi_ref, c2r_ref, dwk_ref, dwv_ref,
                 kout_ref, vout_ref, zr_ref, zi_ref):
    l = pl.program_id(0)

    @pl.when(l == 0)
    def _init():
        pr = pr_ref[...]
        zr_ref[...] = _mm(pr, pwr_ref[...])
        zi_ref[...] = _mm(pr, pwi_ref[...])

    zr = zr_ref[...]
    zi = zi_ref[...]
    cqr = cqr_ref[0]; cqi = cqi_ref[0]
    ckr = ckr_ref[0]; cki = cki_ref[0]
    cvr = cvr_ref[0]; cvi = cvi_ref[0]
    cor = cor_ref[0]; coi = coi_ref[0]
    qr = _mm(zr, cqr) - _mm(zi, cqi); qi = _mm(zr, cqi) + _mm(zi, cqr)
    kr = _mm(zr, ckr) - _mm(zi, cki); ki = _mm(zr, cki) + _mm(zi, ckr)
    vr = _mm(zr, cvr) - _mm(zi, cvi); vi = _mm(zr, cvi) + _mm(zi, cvr)

    n = B * (S // P)
    r_iota = jax.lax.broadcasted_iota(jnp.int32, (n, n), 0)
    c_iota = jax.lax.broadcasted_iota(jnp.int32, (n, n), 1)
    bmask = (r_iota // (S // P)) == (c_iota // (S // P))
    scale = 1.0 / math.sqrt(DH_E)
    a_list, vr_list, vi_list = [], [], []
    for h in range(NH):
        mh = _head_mask(C, DH_E, h)
        s = (_mm_t(qr * mh, kr) + _mm_t(qi * mh, ki)) * scale
        s = jnp.where(bmask, s, -1e30)
        a_list.append(_softmax_rows(s))
        vr_list.append(vr * mh)
        vi_list.append(vi * mh)
    A = jnp.concatenate(a_list, axis=1)
    ar = _mm(A, jnp.concatenate(vr_list, axis=0))
    ai = _mm(A, jnp.concatenate(vi_list, axis=0))
    orr = _mm(ar, cor) - _mm(ai, coi)
    oii = _mm(ar, coi) + _mm(ai, cor)
    zr = _ln_f32(zr + orr)
    zi = _ln_f32(zi + oii)
    zr_ref[...] = zr
    zi_ref[...] = zi

    @pl.when(l == L - 1)
    def _fini():
        mem = _mm(jnp.concatenate([zr, zi], axis=1), c2r_ref[...])
        kout_ref[...] = _mm(mem, dwk_ref[...])
        vout_ref[...] = _mm(mem, dwv_ref[...])


def _dec_kernel(ids_ref, k_ref, v_ref, emb_ref, wq_ref, wo_ref,
                ff1_ref, ff2_ref, outw_ref, out_ref):
    ids = ids_ref[0]
    oh = (jax.lax.broadcasted_iota(jnp.int32, (V, DEC_TB), 0) == ids)
    y = _mm_tt(oh.astype(CDT), emb_ref[...])
    yn = _ln_f32(y)
    q = _mm(yn, wq_ref[...])
    Kb = k_ref[0]
    Vb = v_ref[0]
    scale = 1.0 / math.sqrt(DH_D)
    a_list, v_list = [], []
    for h in range(NH):
        mh = _head_mask(DM, DH_D, h)
        s = _mm_t(q * mh, Kb) * scale
        a_list.append(_softmax_rows(s))
        v_list.append(Vb * mh)
    A = jnp.concatenate(a_list, axis=1)
    attn = _mm(A, jnp.concatenate(v_list, axis=0))
    y = y + _mm(attn, wo_ref[...])
    y2 = _ln_f32(y)
    y = y + _mm(_gelu(_mm(y2, ff1_ref[...])), ff2_ref[...])
    out_ref[0] = _mm(_ln_f32(y), outw_ref[...])


def _const_spec(shape):
    nd = len(shape)
    return pl.BlockSpec(shape, lambda *args: (0,) * nd)


@jax.jit
def kernel(byte_seq, target_byte_seq, emb_byte, enc_Wq, enc_Wk, enc_Wv, enc_Wo,
           enc_ff1, enc_ff2, proj_Wr, proj_Wi, cq_r, cq_i, ck_r, ck_i, cv_r,
           cv_i, co_r, co_i, c2r_W, dec_emb, dec_Wq, dec_Wk, dec_Wv, dec_Wo,
           dec_ff1, dec_ff2, out_W):
    f16 = lambda w: w.astype(CDT)
    Np = S // P
    n = B * Np

    n_enc = (B * S) // ENC_TB
    ids_enc = byte_seq.astype(jnp.int32).reshape(n_enc, 1, ENC_TB)
    patch = pl.pallas_call(
        _enc_kernel,
        grid=(n_enc,),
        in_specs=[
            pl.BlockSpec((1, 1, ENC_TB), lambda i: (i, 0, 0)),
            _const_spec((V, H)), _const_spec((H, H)), _const_spec((H, H)),
            _const_spec((H, H)), _const_spec((H, H)),
            _const_spec((H, FF)), _const_spec((FF, H)),
        ],
        out_specs=pl.BlockSpec((1, ENC_NPB, H), lambda i: (i, 0, 0)),
        out_shape=jax.ShapeDtypeStruct((n_enc, ENC_NPB, H), jnp.float32),
        compiler_params=pltpu.CompilerParams(
            dimension_semantics=("arbitrary",)),
    )(ids_enc, f16(emb_byte), f16(enc_Wq), f16(enc_Wk), f16(enc_Wv),
      f16(enc_Wo), f16(enc_ff1), f16(enc_ff2))
    patch = patch.reshape(n, H)

    lspec = pl.BlockSpec((1, C, C), lambda l: (l, 0, 0))
    kv = pl.pallas_call(
        _cplx_kernel,
        grid=(L,),
        in_specs=[
            _const_spec((n, H)), _const_spec((H, C)), _const_spec((H, C)),
            lspec, lspec, lspec, lspec, lspec, lspec, lspec, lspec,
            _const_spec((2 * C, DM)), _const_spec((DM, DM)),
            _const_spec((DM, DM)),
        ],
        out_specs=[_const_spec((n, DM)), _const_spec((n, DM))],
        out_shape=[jax.ShapeDtypeStruct((n, DM), jnp.float32),
                   jax.ShapeDtypeStruct((n, DM), jnp.float32)],
        scratch_shapes=[pltpu.VMEM((n, C), jnp.float32),
                        pltpu.VMEM((n, C), jnp.float32)],
        compiler_params=pltpu.CompilerParams(
            dimension_semantics=("arbitrary",)),
    )(patch, f16(proj_Wr), f16(proj_Wi), f16(cq_r), f16(cq_i), f16(ck_r),
      f16(ck_i), f16(cv_r), f16(cv_i), f16(co_r), f16(co_i), f16(c2r_W),
      f16(dec_Wk), f16(dec_Wv))
    K, Vv = kv
    K = K.reshape(B, Np, DM)
    Vv = Vv.reshape(B, Np, DM)

    n_dec = T // DEC_TB
    ids_dec = target_byte_seq.astype(jnp.int32).reshape(B * n_dec, 1, DEC_TB)
    logits = pl.pallas_call(
        _dec_kernel,
        grid=(B, n_dec),
        in_specs=[
            pl.BlockSpec((1, 1, DEC_TB), lambda b, t: (b * n_dec + t, 0, 0)),
            pl.BlockSpec((1, Np, DM), lambda b, t: (b, 0, 0)),
            pl.BlockSpec((1, Np, DM), lambda b, t: (b, 0, 0)),
            _const_spec((V, DM)), _const_spec((DM, DM)), _const_spec((DM, DM)),
            _const_spec((DM, FF)), _const_spec((FF, DM)),
            _const_spec((DM, V)),
        ],
        out_specs=pl.BlockSpec((1, DEC_TB, V), lambda b, t: (b, t, 0)),
        out_shape=jax.ShapeDtypeStruct((B, T, V), jnp.float32),
        compiler_params=pltpu.CompilerParams(
            dimension_semantics=("arbitrary", "arbitrary")),
    )(ids_dec, K, Vv, f16(dec_emb), f16(dec_Wq), f16(dec_Wo), f16(dec_ff1),
      f16(dec_ff2), f16(out_W))
    return logits


# speedup vs baseline: 1.5822x; 1.0611x over previous
import functools
import math

import jax
import jax.numpy as jnp
from jax.experimental import pallas as pl
from jax.experimental.pallas import tpu as pltpu

B = 8; S = 2048; T = 2048; H = 512; C = 512; NH = 8; DM = 768; L = 6; P = 128; V = 256; FF = 2048

CDT = jnp.bfloat16
ACC = jnp.float32

ENC_TB = 512
ENC_NPB = ENC_TB // P
DEC_TB = 1024
DH_E = H // NH
DH_D = DM // NH
NP = S // P


def _mm(a, b):
    return jax.lax.dot_general(a.astype(CDT), b.astype(CDT),
                               (((1,), (0,)), ((), ())),
                               preferred_element_type=ACC)


def _mm_t(a, b):
    return jax.lax.dot_general(a.astype(CDT), b.astype(CDT),
                               (((1,), (1,)), ((), ())),
                               preferred_element_type=ACC)


def _mm_tt(a, b):
    return jax.lax.dot_general(a.astype(CDT), b.astype(CDT),
                               (((0,), (0,)), ((), ())),
                               preferred_element_type=ACC)


def _ln_f32(x):
    m = jnp.mean(x, axis=-1, keepdims=True)
    v = jnp.mean((x - m) * (x - m), axis=-1, keepdims=True)
    return (x - m) / jnp.sqrt(v + 1e-5)


def _gelu(x):
    c = math.sqrt(2.0 / math.pi)
    return 0.5 * x * (1.0 + jnp.tanh(c * (x + 0.044715 * (x * x * x))))


def _head_mask(width, dh, h):
    lane = jax.lax.broadcasted_iota(jnp.int32, (1, width), 1)
    return (lane // dh == h).astype(ACC)


def _softmax_rows(s):
    m = jnp.max(s, axis=-1, keepdims=True)
    e = jnp.exp(s - m)
    return e / jnp.sum(e, axis=-1, keepdims=True)


def _enc_kernel(ids_ref, emb_ref, wq_ref, wk_ref, wv_ref, wo_ref,
                ff1_ref, ff2_ref, out_ref):
    ids = ids_ref[0]
    oh = (jax.lax.broadcasted_iota(jnp.int32, (V, ENC_TB), 0) == ids)
    x = _mm_tt(oh.astype(CDT), emb_ref[...])
    h1 = _ln_f32(x)
    q = _mm(h1, wq_ref[...])
    k = _mm(h1, wk_ref[...])
    v = _mm(h1, wv_ref[...])
    qm = [(q * _head_mask(H, DH_E, h)).astype(CDT) for h in range(NH)]
    vm = [(v * _head_mask(H, DH_E, h)).astype(CDT) for h in range(NH)]
    kb = k.astype(CDT)
    scale = 1.0 / math.sqrt(DH_E)
    rows = []
    for p in range(ENC_NPB):
        sl = slice(p * P, (p + 1) * P)
        kp = kb[sl]
        a_list = [_softmax_rows(_mm_t(qm[h][sl], kp) * scale) for h in range(NH)]
        A = jnp.concatenate(a_list, axis=1)
        Vhat = jnp.concatenate([vm[h][sl] for h in range(NH)], axis=0)
        rows.append(_mm(A, Vhat))
    attn = jnp.concatenate(rows, axis=0)
    h = x + _mm(attn, wo_ref[...])
    h2 = _ln_f32(h)
    h = h + _mm(_gelu(_mm(h2, ff1_ref[...]).astype(CDT)), ff2_ref[...])
    pooled = jnp.concatenate(
        [jnp.mean(h[p * P:(p + 1) * P], axis=0, keepdims=True)
         for p in range(ENC_NPB)], axis=0)
    out_ref[0] = pooled


def _cplx_kernel(pr_ref, pwr_ref, pwi_ref, cqr_ref, cqi_ref, ckr_ref, cki_ref,
                 cvr_ref, cvi_ref, cor_ref, coi_ref, c2r_ref, dwk_ref, dwv_ref,
                 kout_ref, vout_ref, zr_ref, zi_ref):
    l = pl.program_id(0)

    @pl.when(l == 0)
    def _init():
        pr = pr_ref[...]
        zr_ref[...] = _mm(pr, pwr_ref[...])
        zi_ref[...] = _mm(pr, pwi_ref[...])

    zr = zr_ref[...]
    zi = zi_ref[...]
    cqr = cqr_ref[0]; cqi = cqi_ref[0]
    ckr = ckr_ref[0]; cki = cki_ref[0]
    cvr = cvr_ref[0]; cvi = cvi_ref[0]
    cor = cor_ref[0]; coi = coi_ref[0]
    qr = _mm(zr, cqr) - _mm(zi, cqi); qi = _mm(zr, cqi) + _mm(zi, cqr)
    kr = _mm(zr, ckr) - _mm(zi, cki); ki = _mm(zr, cki) + _mm(zi, ckr)
    vr = _mm(zr, cvr) - _mm(zi, cvi); vi = _mm(zr, cvi) + _mm(zi, cvr)

    n = B * (S // P)
    r_iota = jax.lax.broadcasted_iota(jnp.int32, (n, n), 0)
    c_iota = jax.lax.broadcasted_iota(jnp.int32, (n, n), 1)
    bmask = (r_iota // (S // P)) == (c_iota // (S // P))
    scale = 1.0 / math.sqrt(DH_E)
    krb = kr.astype(CDT); kib = ki.astype(CDT)
    a_list, vr_list, vi_list = [], [], []
    for h in range(NH):
        mh = _head_mask(C, DH_E, h)
        s = (_mm_t((qr * mh).astype(CDT), krb) +
             _mm_t((qi * mh).astype(CDT), kib)) * scale
        s = jnp.where(bmask, s, -1e30)
        a_list.append(_softmax_rows(s))
        vr_list.append((vr * mh).astype(CDT))
        vi_list.append((vi * mh).astype(CDT))
    A = jnp.concatenate(a_list, axis=1)
    ar = _mm(A, jnp.concatenate(vr_list, axis=0))
    ai = _mm(A, jnp.concatenate(vi_list, axis=0))
    orr = _mm(ar, cor) - _mm(ai, coi)
    oii = _mm(ar, coi) + _mm(ai, cor)
    zr = _ln_f32(zr + orr)
    zi = _ln_f32(zi + oii)
    zr_ref[...] = zr
    zi_ref[...] = zi

    @pl.when(l == L - 1)
    def _fini():
        mem = _mm(jnp.concatenate([zr, zi], axis=1), c2r_ref[...])
        kout_ref[...] = _mm(mem, dwk_ref[...])
        vout_ref[...] = _mm(mem, dwv_ref[...])


def _dec_kernel(ids_ref, k_ref, v_ref, emb_ref, wq_ref, wo_ref,
                ff1_ref, ff2_ref, outw_ref, out_ref):
    ids = ids_ref[0]
    oh = (jax.lax.broadcasted_iota(jnp.int32, (V, DEC_TB), 0) == ids)
    y = _mm_tt(oh.astype(CDT), emb_ref[...])
    yn = _ln_f32(y)
    q = _mm(yn, wq_ref[...])
    Kb = k_ref[0]
    Vb = v_ref[0]
    scale = 1.0 / math.sqrt(DH_D)
    Kbb = Kb.astype(CDT)
    s_list, v_list = [], []
    for h in range(NH):
        mh = _head_mask(DM, DH_D, h)
        s_list.append(_mm_t((q * mh).astype(CDT), Kbb) * scale)
        v_list.append((Vb * mh).astype(CDT))
    sc = jnp.concatenate(s_list, axis=1)
    m = jnp.max(sc, axis=-1, keepdims=True)
    e = jnp.exp(sc - m)
    gr = jax.lax.broadcasted_iota(jnp.int32, (NH * NP, NH * NP), 0) // NP
    gc = jax.lax.broadcasted_iota(jnp.int32, (NH * NP, NH * NP), 1) // NP
    G = (gr == gc).astype(jnp.float32)
    denom = jax.lax.dot_general(e, G, (((1,), (0,)), ((), ())),
                                preferred_element_type=jnp.float32)
    A = e / denom
    attn = _mm(A, jnp.concatenate(v_list, axis=0))
    y = y + _mm(attn, wo_ref[...])
    y2 = _ln_f32(y)
    y = y + _mm(_gelu(_mm(y2, ff1_ref[...]).astype(CDT)), ff2_ref[...])
    out_ref[0] = _mm(_ln_f32(y), outw_ref[...])


def _const_spec(shape):
    nd = len(shape)
    return pl.BlockSpec(shape, lambda *args: (0,) * nd)


@jax.jit
def kernel(byte_seq, target_byte_seq, emb_byte, enc_Wq, enc_Wk, enc_Wv, enc_Wo,
           enc_ff1, enc_ff2, proj_Wr, proj_Wi, cq_r, cq_i, ck_r, ck_i, cv_r,
           cv_i, co_r, co_i, c2r_W, dec_emb, dec_Wq, dec_Wk, dec_Wv, dec_Wo,
           dec_ff1, dec_ff2, out_W):
    f16 = lambda w: w.astype(CDT)
    Np = S // P
    n = B * Np

    n_enc = (B * S) // ENC_TB
    ids_enc = byte_seq.astype(jnp.int32).reshape(n_enc, 1, ENC_TB)
    patch = pl.pallas_call(
        _enc_kernel,
        grid=(n_enc,),
        in_specs=[
            pl.BlockSpec((1, 1, ENC_TB), lambda i: (i, 0, 0)),
            _const_spec((V, H)), _const_spec((H, H)), _const_spec((H, H)),
            _const_spec((H, H)), _const_spec((H, H)),
            _const_spec((H, FF)), _const_spec((FF, H)),
        ],
        out_specs=pl.BlockSpec((1, ENC_NPB, H), lambda i: (i, 0, 0)),
        out_shape=jax.ShapeDtypeStruct((n_enc, ENC_NPB, H), jnp.float32),
        compiler_params=pltpu.CompilerParams(
            dimension_semantics=("arbitrary",)),
    )(ids_enc, f16(emb_byte), f16(enc_Wq), f16(enc_Wk), f16(enc_Wv),
      f16(enc_Wo), f16(enc_ff1), f16(enc_ff2))
    patch = patch.reshape(n, H)

    lspec = pl.BlockSpec((1, C, C), lambda l: (l, 0, 0))
    kv = pl.pallas_call(
        _cplx_kernel,
        grid=(L,),
        in_specs=[
            _const_spec((n, H)), _const_spec((H, C)), _const_spec((H, C)),
            lspec, lspec, lspec, lspec, lspec, lspec, lspec, lspec,
            _const_spec((2 * C, DM)), _const_spec((DM, DM)),
            _const_spec((DM, DM)),
        ],
        out_specs=[_const_spec((n, DM)), _const_spec((n, DM))],
        out_shape=[jax.ShapeDtypeStruct((n, DM), jnp.float32),
                   jax.ShapeDtypeStruct((n, DM), jnp.float32)],
        scratch_shapes=[pltpu.VMEM((n, C), jnp.float32),
                        pltpu.VMEM((n, C), jnp.float32)],
        compiler_params=pltpu.CompilerParams(
            dimension_semantics=("arbitrary",)),
    )(patch, f16(proj_Wr), f16(proj_Wi), f16(cq_r), f16(cq_i), f16(ck_r),
      f16(ck_i), f16(cv_r), f16(cv_i), f16(co_r), f16(co_i), f16(c2r_W),
      f16(dec_Wk), f16(dec_Wv))
    K, Vv = kv
    K = K.reshape(B, Np, DM)
    Vv = Vv.reshape(B, Np, DM)

    n_dec = T // DEC_TB
    ids_dec = target_byte_seq.astype(jnp.int32).reshape(B * n_dec, 1, DEC_TB)
    logits = pl.pallas_call(
        _dec_kernel,
        grid=(B, n_dec),
        in_specs=[
            pl.BlockSpec((1, 1, DEC_TB), lambda b, t: (b * n_dec + t, 0, 0)),
            pl.BlockSpec((1, Np, DM), lambda b, t: (b, 0, 0)),
            pl.BlockSpec((1, Np, DM), lambda b, t: (b, 0, 0)),
            _const_spec((V, DM)), _const_spec((DM, DM)), _const_spec((DM, DM)),
            _const_spec((DM, FF)), _const_spec((FF, DM)),
            _const_spec((DM, V)),
        ],
        out_specs=pl.BlockSpec((1, DEC_TB, V), lambda b, t: (b, t, 0)),
        out_shape=jax.ShapeDtypeStruct((B, T, V), jnp.float32),
        compiler_params=pltpu.CompilerParams(
            dimension_semantics=("arbitrary", "arbitrary")),
    )(ids_dec, K, Vv, f16(dec_emb), f16(dec_Wq), f16(dec_Wo), f16(dec_ff1),
      f16(dec_ff2), f16(out_W))
    return logits
